# baseline (speedup 1.0000x reference)
import functools

import jax
import jax.numpy as jnp
from jax import lax
from jax.experimental import pallas as pl
from jax.experimental.pallas import tpu as pltpu
from jax.experimental.pallas import tpu_sc as plsc

VOCAB = 1000000
HIDDEN = 64


_ROWS_BLK = 8000


def _transform_body(x_ref, wt_ref, b_ref, o_ref):
    o_ref[...] = (
        jnp.dot(x_ref[...], wt_ref[...], preferred_element_type=jnp.float32)
        + b_ref[...]
    )


def _transform_table(emb_table, Wt, b2):
    return pl.pallas_call(
        _transform_body,
        grid=(VOCAB // _ROWS_BLK,),
        in_specs=[
            pl.BlockSpec((_ROWS_BLK, HIDDEN), lambda i: (i, 0)),
            pl.BlockSpec((HIDDEN, HIDDEN), lambda i: (0, 0)),
            pl.BlockSpec((1, HIDDEN), lambda i: (0, 0)),
        ],
        out_specs=pl.BlockSpec((_ROWS_BLK, HIDDEN), lambda i: (i, 0)),
        out_shape=jax.ShapeDtypeStruct((VOCAB, HIDDEN), jnp.float32),
    )(emb_table, Wt, b2)


_CHUNK = 128
_NW = 32
_NC = 2


def _make_gather(n_chunks_total):
    chunks_per_w = n_chunks_total // _NW
    mesh = plsc.VectorSubcoreMesh(core_axis_name="c", subcore_axis_name="s")

    @functools.partial(
        pl.kernel,
        mesh=mesh,
        compiler_params=pltpu.CompilerParams(use_tc_tiling_on_sc=False),
        out_type=jax.ShapeDtypeStruct((n_chunks_total * _CHUNK, HIDDEN), jnp.float32),
        scratch_types=[
            pltpu.VMEM((chunks_per_w, _CHUNK), jnp.int32),
            pltpu.VMEM((2, _CHUNK, HIDDEN), jnp.float32),
            pltpu.SemaphoreType.DMA,
            pltpu.SemaphoreType.DMA,
        ],
    )
    def gather_k(table_hbm, idx_hbm, out_hbm, idx_v, rows_v, g_sem, s_sem):
        wid = lax.axis_index("s") * _NC + lax.axis_index("c")
        base_chunk = wid * chunks_per_w
        pltpu.sync_copy(idx_hbm.at[pl.ds(base_chunk, chunks_per_w)], idx_v)

        def body(j, _):
            slot = lax.rem(j, 2)
            pltpu.async_copy(
                table_hbm.at[idx_v.at[j]], rows_v.at[slot], g_sem
            ).wait()
            pltpu.async_copy(
                rows_v.at[slot],
                out_hbm.at[pl.ds((base_chunk + j) * _CHUNK, _CHUNK)],
                s_sem,
            ).wait()
            return 0

        lax.fori_loop(0, chunks_per_w, body, 0)

    return gather_k


def kernel(input_ids, emb_table, W, b):
    B, L = input_ids.shape
    n = B * L
    ids2 = input_ids.reshape(n // _CHUNK, _CHUNK).astype(jnp.int32)
    table_t = _transform_table(emb_table, W.T, b.reshape(1, HIDDEN))
    out = _make_gather(n // _CHUNK)(table_t, ids2)
    return out.reshape(B, L, HIDDEN)


# speedup vs baseline: 1.1898x; 1.1898x over previous
import functools

import jax
import jax.numpy as jnp
from jax import lax
from jax.experimental import pallas as pl
from jax.experimental.pallas import tpu as pltpu
from jax.experimental.pallas import tpu_sc as plsc

VOCAB = 1000000
HALF = VOCAB // 2
HIDDEN = 64


_T_BLK = 4000


def _transform_body(xa_ref, xb_ref, wt_ref, b_ref, o_ref):
    ya = jnp.dot(xa_ref[...], wt_ref[...], preferred_element_type=jnp.float32)
    yb = jnp.dot(xb_ref[...], wt_ref[...], preferred_element_type=jnp.float32)
    o_ref[...] = jnp.concatenate([ya, yb], axis=1) + b_ref[...]


def _transform_table(emb_table, Wt, b2):
    nblk_half = HALF // _T_BLK
    return pl.pallas_call(
        _transform_body,
        grid=(nblk_half,),
        in_specs=[
            pl.BlockSpec((_T_BLK, HIDDEN), lambda i: (i, 0)),
            pl.BlockSpec((_T_BLK, HIDDEN), lambda i, n=nblk_half: (i + n, 0)),
            pl.BlockSpec((HIDDEN, HIDDEN), lambda i: (0, 0)),
            pl.BlockSpec((1, 2 * HIDDEN), lambda i: (0, 0)),
        ],
        out_specs=pl.BlockSpec((_T_BLK, 2 * HIDDEN), lambda i: (i, 0)),
        out_shape=jax.ShapeDtypeStruct((HALF, 2 * HIDDEN), jnp.float32),
    )(emb_table, emb_table, Wt, b2)


_NW = 32
_NC = 2
_ROWS_PER_CHUNK = 64
_NBUF = 4


def _make_gather(n_chunks_total):
    chunks_per_w = n_chunks_total // _NW
    mesh = plsc.VectorSubcoreMesh(core_axis_name="c", subcore_axis_name="s")
    n_rows32 = n_chunks_total * _ROWS_PER_CHUNK * 2

    @functools.partial(
        pl.kernel,
        mesh=mesh,
        compiler_params=pltpu.CompilerParams(use_tc_tiling_on_sc=False),
        out_type=jax.ShapeDtypeStruct((n_rows32, 32), jnp.float32),
        scratch_types=[
            pltpu.VMEM((chunks_per_w, 128), jnp.int32),
            pltpu.VMEM((_NBUF, 128, 32), jnp.float32),
            pltpu.SemaphoreType.DMA,
            pltpu.SemaphoreType.DMA,
        ],
    )
    def gather_k(table32_hbm, idx_hbm, out_hbm, idx_v, rows_v, g_sem, s_sem):
        wid = lax.axis_index("s") * _NC + lax.axis_index("c")
        base_chunk = wid * chunks_per_w
        pltpu.sync_copy(idx_hbm.at[pl.ds(base_chunk, chunks_per_w)], idx_v)

        def block(g, _):
            gcps = []
            for t in range(_NBUF):
                gcps.append(
                    pltpu.async_copy(
                        table32_hbm.at[idx_v.at[g + t]], rows_v.at[t], g_sem
                    )
                )
            scps = []
            for t in range(_NBUF):
                gcps[t].wait()
                scps.append(
                    pltpu.async_copy(
                        rows_v.at[t],
                        out_hbm.at[pl.ds((base_chunk + g + t) * 128, 128)],
                        s_sem,
                    )
                )
            for t in range(_NBUF):
                scps[t].wait()
            return 0

        lax.fori_loop(0, chunks_per_w // _NBUF, lambda i, c: block(i * _NBUF, c), 0)

    return gather_k


_U_BLK = 4096


def _unpack_body(x_ref, o_ref):
    g = pl.program_id(0)

    @pl.when(g == 0)
    def _():
        o_ref[...] = x_ref[:, :HIDDEN]

    @pl.when(g == 1)
    def _():
        o_ref[...] = x_ref[:, HIDDEN:]


def _unpack(g128, n_rows):
    n_lines = n_rows // 2
    nblk = n_lines // _U_BLK
    return pl.pallas_call(
        _unpack_body,
        grid=(2, nblk),
        in_specs=[pl.BlockSpec((_U_BLK, 2 * HIDDEN), lambda g, j: (j, 0))],
        out_specs=pl.BlockSpec((_U_BLK, HIDDEN), lambda g, j, n=nblk: (g * n + j, 0)),
        out_shape=jax.ShapeDtypeStruct((n_rows, HIDDEN), jnp.float32),
    )(g128)


def kernel(input_ids, emb_table, W, b):
    B, L = input_ids.shape
    n = B * L

    b2 = jnp.concatenate([b, b]).reshape(1, 2 * HIDDEN)
    table128 = _transform_table(emb_table, W.T, b2)
    table32 = table128.reshape(2 * VOCAB, 32)

    ids_flat = input_ids.reshape(-1).astype(jnp.int32)
    base = (ids_flat % HALF) * 4 + (ids_flat // HALF) * 2
    b1 = base[: n // 2]
    b2_ = base[n // 2 :]
    idx = jnp.stack([b1, b1 + 1, b2_, b2_ + 1], axis=1).reshape(
        n // _ROWS_PER_CHUNK, 128
    )

    out32 = _make_gather(n // _ROWS_PER_CHUNK)(table32, idx)
    out = _unpack(out32.reshape(n // 2, 2 * HIDDEN), n)
    return out.reshape(B, L, HIDDEN)


# speedup vs baseline: 1.2985x; 1.0914x over previous
import functools

import jax
import jax.numpy as jnp
from jax import lax
from jax.experimental import pallas as pl
from jax.experimental.pallas import tpu as pltpu
from jax.experimental.pallas import tpu_sc as plsc

VOCAB = 1000000
HIDDEN = 64


_T_BLK = 8000


def _transform_body(x_ref, wt_ref, b_ref, o_ref):
    y = (
        jnp.dot(x_ref[...], wt_ref[...], preferred_element_type=jnp.float32)
        + b_ref[...]
    )
    o_ref[...] = jnp.concatenate([y, y], axis=1)


def _transform_table(emb_table, Wt, b2):
    return pl.pallas_call(
        _transform_body,
        grid=(VOCAB // _T_BLK,),
        in_specs=[
            pl.BlockSpec((_T_BLK, HIDDEN), lambda i: (i, 0)),
            pl.BlockSpec((HIDDEN, HIDDEN), lambda i: (0, 0)),
            pl.BlockSpec((1, HIDDEN), lambda i: (0, 0)),
        ],
        out_specs=pl.BlockSpec((_T_BLK, 2 * HIDDEN), lambda i: (i, 0)),
        out_shape=jax.ShapeDtypeStruct((VOCAB, 2 * HIDDEN), jnp.float32),
    )(emb_table, Wt, b2)


_NW = 32
_NC = 2
_CHUNK = 128
_NBUF = 4


def _make_gather(n_chunks_total, n_rows):
    chunks_per_w = n_chunks_total // _NW
    mesh = plsc.VectorSubcoreMesh(core_axis_name="c", subcore_axis_name="s")

    @functools.partial(
        pl.kernel,
        mesh=mesh,
        out_type=jax.ShapeDtypeStruct((n_rows, 2 * HIDDEN), jnp.float32),
        scratch_types=[
            pltpu.VMEM((chunks_per_w, _CHUNK), jnp.int32),
            pltpu.VMEM((_NBUF, _CHUNK, 2 * HIDDEN), jnp.float32),
            pltpu.SemaphoreType.DMA,
            pltpu.SemaphoreType.DMA,
        ],
    )
    def gather_k(table_hbm, idx_hbm, out_hbm, idx_v, rows_v, g_sem, s_sem):
        wid = lax.axis_index("s") * _NC + lax.axis_index("c")
        base_chunk = wid * chunks_per_w
        pltpu.sync_copy(idx_hbm.at[pl.ds(base_chunk, chunks_per_w)], idx_v)

        def block(g, _):
            gcps = []
            for t in range(_NBUF):
                gcps.append(
                    pltpu.async_copy(
                        table_hbm.at[idx_v.at[g + t]], rows_v.at[t], g_sem
                    )
                )
            scps = []
            for t in range(_NBUF):
                gcps[t].wait()
                scps.append(
                    pltpu.async_copy(
                        rows_v.at[t],
                        out_hbm.at[pl.ds((base_chunk + g + t) * _CHUNK, _CHUNK)],
                        s_sem,
                    )
                )
            for t in range(_NBUF):
                scps[t].wait()
            return 0

        lax.fori_loop(0, chunks_per_w // _NBUF, lambda i, c: block(i * _NBUF, c), 0)

    return gather_k


_U_BLK = 8192


def _unpack_body(x_ref, o_ref):
    o_ref[...] = x_ref[:, :HIDDEN]


def _unpack(g128, n_rows):
    return pl.pallas_call(
        _unpack_body,
        grid=(n_rows // _U_BLK,),
        in_specs=[pl.BlockSpec((_U_BLK, 2 * HIDDEN), lambda i: (i, 0))],
        out_specs=pl.BlockSpec((_U_BLK, HIDDEN), lambda i: (i, 0)),
        out_shape=jax.ShapeDtypeStruct((n_rows, HIDDEN), jnp.float32),
    )(g128)


def kernel(input_ids, emb_table, W, b):
    B, L = input_ids.shape
    n = B * L

    table128 = _transform_table(emb_table, W.T, b.reshape(1, HIDDEN))
    idx = input_ids.reshape(n // _CHUNK, _CHUNK).astype(jnp.int32)
    out128 = _make_gather(n // _CHUNK, n)(table128, idx)
    out = _unpack(out128, n)
    return out.reshape(B, L, HIDDEN)


# speedup vs baseline: 1.9988x; 1.5394x over previous
import functools

import jax
import jax.numpy as jnp
from jax import lax
from jax.experimental import pallas as pl
from jax.experimental.pallas import tpu as pltpu
from jax.experimental.pallas import tpu_sc as plsc

VOCAB = 1000000
HIDDEN = 64


_T_BLK = 8192


def _transform_body(xt_ref, w_ref, b_ref, o_ref):
    yt = jnp.dot(w_ref[...], xt_ref[...], preferred_element_type=jnp.float32)
    y = jnp.transpose(yt) + b_ref[...]
    o_ref[...] = jnp.concatenate([y, y], axis=1)


def _transform_table(emb_table_t, W, b_row):
    return pl.pallas_call(
        _transform_body,
        grid=(pl.cdiv(VOCAB, _T_BLK),),
        in_specs=[
            pl.BlockSpec((HIDDEN, _T_BLK), lambda i: (0, i)),
            pl.BlockSpec((HIDDEN, HIDDEN), lambda i: (0, 0)),
            pl.BlockSpec((1, HIDDEN), lambda i: (0, 0)),
        ],
        out_specs=pl.BlockSpec((_T_BLK, 2 * HIDDEN), lambda i: (i, 0)),
        out_shape=jax.ShapeDtypeStruct((VOCAB, 2 * HIDDEN), jnp.float32),
    )(emb_table_t, W, b_row)


_NW = 32
_NC = 2
_CHUNK = 128
_NBUF = 4


def _make_gather(n_rows):
    n_chunks_total = n_rows // _CHUNK
    chunks_per_w = n_chunks_total // _NW
    mesh = plsc.VectorSubcoreMesh(core_axis_name="c", subcore_axis_name="s")

    @functools.partial(
        pl.kernel,
        mesh=mesh,
        out_type=jax.ShapeDtypeStruct((n_rows, 2 * HIDDEN), jnp.float32),
        scratch_types=[
            pltpu.VMEM((chunks_per_w, _CHUNK), jnp.int32),
            pltpu.VMEM((_NBUF, _CHUNK, 2 * HIDDEN), jnp.float32),
            pltpu.SemaphoreType.DMA,
            pltpu.SemaphoreType.DMA,
        ],
    )
    def gather_k(table_hbm, idx_hbm, out_hbm, idx_v, rows_v, g_sem, s_sem):
        wid = lax.axis_index("s") * _NC + lax.axis_index("c")
        base_chunk = wid * chunks_per_w
        pltpu.sync_copy(idx_hbm.at[pl.ds(base_chunk, chunks_per_w)], idx_v)

        def block(g, _):
            gcps = []
            for t in range(_NBUF):
                gcps.append(
                    pltpu.async_copy(
                        table_hbm.at[idx_v.at[g + t]], rows_v.at[t], g_sem
                    )
                )
            scps = []
            for t in range(_NBUF):
                gcps[t].wait()
                scps.append(
                    pltpu.async_copy(
                        rows_v.at[t],
                        out_hbm.at[pl.ds((base_chunk + g + t) * _CHUNK, _CHUNK)],
                        s_sem,
                    )
                )
            for t in range(_NBUF):
                scps[t].wait()
            return 0

        lax.fori_loop(0, chunks_per_w // _NBUF, lambda i, c: block(i * _NBUF, c), 0)

    return gather_k


def _unpack_body(x_ref, o_ref):
    o_ref[...] = jnp.transpose(x_ref[:, :HIDDEN])[None]


def _unpack(lines, L, B):
    return pl.pallas_call(
        _unpack_body,
        grid=(L,),
        in_specs=[pl.BlockSpec((B, 2 * HIDDEN), lambda i: (i, 0))],
        out_specs=pl.BlockSpec((1, HIDDEN, B), lambda i: (i, 0, 0)),
        out_shape=jax.ShapeDtypeStruct((L, HIDDEN, B), jnp.float32),
    )(lines)


def kernel(input_ids, emb_table, W, b):
    B, L = input_ids.shape
    n = B * L

    table128 = _transform_table(emb_table.T, W, b.reshape(1, HIDDEN))
    idx = input_ids.T.reshape(n // _CHUNK, _CHUNK).astype(jnp.int32)
    lines = _make_gather(n)(table128, idx)
    out_t = _unpack(lines, L, B)
    return jnp.transpose(out_t, (2, 0, 1))


# speedup vs baseline: 2.1243x; 1.0628x over previous
import functools

import jax
import jax.numpy as jnp
from jax import lax
from jax.experimental import pallas as pl
from jax.experimental.pallas import tpu as pltpu
from jax.experimental.pallas import tpu_sc as plsc

VOCAB = 1000000
HIDDEN = 64


_T_BLK = 8192


def _transform_body(xt_ref, w_ref, b_ref, o_ref):
    yt = jnp.dot(w_ref[...], xt_ref[...], preferred_element_type=jnp.float32)
    y = jnp.transpose(yt) + b_ref[...]
    o_ref[...] = jnp.concatenate([y, y], axis=1)


def _transform_table(emb_table_t, W, b_row):
    return pl.pallas_call(
        _transform_body,
        grid=(pl.cdiv(VOCAB, _T_BLK),),
        in_specs=[
            pl.BlockSpec((HIDDEN, _T_BLK), lambda i: (0, i)),
            pl.BlockSpec((HIDDEN, HIDDEN), lambda i: (0, 0)),
            pl.BlockSpec((1, HIDDEN), lambda i: (0, 0)),
        ],
        out_specs=pl.BlockSpec((_T_BLK, 2 * HIDDEN), lambda i: (i, 0)),
        out_shape=jax.ShapeDtypeStruct((VOCAB, 2 * HIDDEN), jnp.float32),
    )(emb_table_t, W, b_row)


_NW = 32
_NC = 2
_CHUNK = 128
_NBUF = 4
_NGROUPS = 5


def _make_gather(n_rows, group):
    n_chunks_total = n_rows // _CHUNK
    chunks_per_w = n_chunks_total // _NW
    group_base = group * n_chunks_total
    mesh = plsc.VectorSubcoreMesh(core_axis_name="c", subcore_axis_name="s")

    @functools.partial(
        pl.kernel,
        mesh=mesh,
        out_type=jax.ShapeDtypeStruct((n_rows, 2 * HIDDEN), jnp.float32),
        scratch_types=[
            pltpu.VMEM((chunks_per_w, _CHUNK), jnp.int32),
            pltpu.VMEM((_NBUF, _CHUNK, 2 * HIDDEN), jnp.float32),
            pltpu.SemaphoreType.DMA,
            pltpu.SemaphoreType.DMA,
        ],
    )
    def gather_k(table_hbm, idx_hbm, out_hbm, idx_v, rows_v, g_sem, s_sem):
        wid = lax.axis_index("s") * _NC + lax.axis_index("c")
        base_chunk = wid * chunks_per_w
        pltpu.sync_copy(
            idx_hbm.at[pl.ds(group_base + base_chunk, chunks_per_w)], idx_v
        )

        def block(g, _):
            gcps = []
            for t in range(_NBUF):
                gcps.append(
                    pltpu.async_copy(
                        table_hbm.at[idx_v.at[g + t]], rows_v.at[t], g_sem
                    )
                )
            scps = []
            for t in range(_NBUF):
                gcps[t].wait()
                scps.append(
                    pltpu.async_copy(
                        rows_v.at[t],
                        out_hbm.at[pl.ds((base_chunk + g + t) * _CHUNK, _CHUNK)],
                        s_sem,
                    )
                )
            for t in range(_NBUF):
                scps[t].wait()
            return 0

        lax.fori_loop(0, chunks_per_w // _NBUF, lambda i, c: block(i * _NBUF, c), 0)

    return gather_k


def _unpack_body(x_ref, o_ref):
    o_ref[...] = jnp.transpose(x_ref[:, :HIDDEN])[None]


def _unpack_body_alias(x_ref, _, o_ref):
    o_ref[...] = jnp.transpose(x_ref[:, :HIDDEN])[None]


def _unpack_group(lines_g, out_prev, l_base, l_cnt, L, B):
    out_shape = jax.ShapeDtypeStruct((L, HIDDEN, B), jnp.float32)
    out_spec = pl.BlockSpec((1, HIDDEN, B), lambda i, l0=l_base: (l0 + i, 0, 0))
    in_spec = pl.BlockSpec((B, 2 * HIDDEN), lambda i: (i, 0))
    if out_prev is None:
        return pl.pallas_call(
            _unpack_body,
            grid=(l_cnt,),
            in_specs=[in_spec],
            out_specs=out_spec,
            out_shape=out_shape,
        )(lines_g)
    return pl.pallas_call(
        _unpack_body_alias,
        grid=(l_cnt,),
        in_specs=[in_spec, pl.BlockSpec(memory_space=pl.ANY)],
        out_specs=out_spec,
        out_shape=out_shape,
        input_output_aliases={1: 0},
    )(lines_g, out_prev)


def kernel(input_ids, emb_table, W, b):
    B, L = input_ids.shape
    n = B * L

    table128 = _transform_table(emb_table.T, W, b.reshape(1, HIDDEN))
    idx = input_ids.T.reshape(n // _CHUNK, _CHUNK).astype(jnp.int32)

    rows_per_group = n // _NGROUPS
    l_per_group = L // _NGROUPS
    out_t = None
    for g in range(_NGROUPS):
        lines_g = _make_gather(rows_per_group, g)(table128, idx)
        out_t = _unpack_group(lines_g, out_t, g * l_per_group, l_per_group, L, B)
    return jnp.transpose(out_t, (2, 0, 1))


# speedup vs baseline: 2.4120x; 1.1354x over previous
import functools

import jax
import jax.numpy as jnp
from jax import lax
from jax.experimental import pallas as pl
from jax.experimental.pallas import tpu as pltpu
from jax.experimental.pallas import tpu_sc as plsc

VOCAB = 1000000
HIDDEN = 64


_T_BLK = 8192


def _transform_body(xt_ref, w_ref, b_ref, o_ref):
    y = lax.dot_general(
        xt_ref[...],
        w_ref[...],
        dimension_numbers=(((0,), (1,)), ((), ())),
        preferred_element_type=jnp.float32,
    ) + b_ref[...]
    o_ref[...] = jnp.concatenate([y, y], axis=1)


def _transform_table(emb_table_t, W, b_row):
    return pl.pallas_call(
        _transform_body,
        grid=(pl.cdiv(VOCAB, _T_BLK),),
        in_specs=[
            pl.BlockSpec((HIDDEN, _T_BLK), lambda i: (0, i)),
            pl.BlockSpec((HIDDEN, HIDDEN), lambda i: (0, 0)),
            pl.BlockSpec((1, HIDDEN), lambda i: (0, 0)),
        ],
        out_specs=pl.BlockSpec((_T_BLK, 2 * HIDDEN), lambda i: (i, 0)),
        out_shape=jax.ShapeDtypeStruct((VOCAB, 2 * HIDDEN), jnp.float32),
    )(emb_table_t, W, b_row)


_NW = 32
_NC = 2
_CHUNK = 128
_NBUF = 2
_NGROUPS = 5


def _make_gather(n_rows, group):
    n_chunks_total = n_rows // _CHUNK
    chunks_per_w = n_chunks_total // _NW
    group_base = group * n_chunks_total
    mesh = plsc.VectorSubcoreMesh(core_axis_name="c", subcore_axis_name="s")

    @functools.partial(
        pl.kernel,
        mesh=mesh,
        out_type=jax.ShapeDtypeStruct((n_rows // 2, 2 * HIDDEN), jnp.float32),
        scratch_types=[
            pltpu.VMEM((chunks_per_w, _CHUNK), jnp.int32),
            pltpu.VMEM((_NBUF, _CHUNK, 2 * HIDDEN), jnp.float32),
            pltpu.VMEM((_NBUF, _CHUNK // 2, 2 * HIDDEN), jnp.float32),
            pltpu.SemaphoreType.DMA,
            pltpu.SemaphoreType.DMA,
        ],
    )
    def gather_k(table_hbm, idx_hbm, out_hbm, idx_v, rows_v, comp_v, g_sem, s_sem):
        wid = lax.axis_index("s") * _NC + lax.axis_index("c")
        base_chunk = wid * chunks_per_w
        pltpu.sync_copy(
            idx_hbm.at[pl.ds(group_base + base_chunk, chunks_per_w)], idx_v
        )

        def compact(t):
            for u in range(_CHUNK // 2):
                for j in range(HIDDEN // 16):
                    comp_v[t, u, pl.ds(16 * j, 16)] = rows_v[
                        t, u, pl.ds(16 * j, 16)
                    ]
                    comp_v[t, u, pl.ds(HIDDEN + 16 * j, 16)] = rows_v[
                        t, 64 + u, pl.ds(16 * j, 16)
                    ]

        def block(g, _):
            gcps = []
            for t in range(_NBUF):
                gcps.append(
                    pltpu.async_copy(
                        table_hbm.at[idx_v.at[g + t]], rows_v.at[t], g_sem
                    )
                )
            scps = []
            for t in range(_NBUF):
                gcps[t].wait()
                compact(t)
                scps.append(
                    pltpu.async_copy(
                        comp_v.at[t],
                        out_hbm.at[
                            pl.ds((base_chunk + g + t) * (_CHUNK // 2), _CHUNK // 2)
                        ],
                        s_sem,
                    )
                )
            for t in range(_NBUF):
                scps[t].wait()
            return 0

        lax.fori_loop(0, chunks_per_w // _NBUF, lambda i, c: block(i * _NBUF, c), 0)

    return gather_k


def _unpack_write(x_ref, o_ref):
    t = jnp.transpose(x_ref[...])
    for c in range(x_ref.shape[0] // (_CHUNK // 2)):
        o_ref[0, :, pl.ds(_CHUNK * c, 64)] = t[:HIDDEN, 64 * c : 64 * c + 64]
        o_ref[0, :, pl.ds(_CHUNK * c + 64, 64)] = t[HIDDEN:, 64 * c : 64 * c + 64]


def _unpack_body(x_ref, o_ref):
    _unpack_write(x_ref, o_ref)


def _unpack_body_alias(x_ref, _, o_ref):
    _unpack_write(x_ref, o_ref)


def _unpack_group(lines_g, out_prev, l_base, l_cnt, L, B):
    out_shape = jax.ShapeDtypeStruct((L, HIDDEN, B), jnp.float32)
    out_spec = pl.BlockSpec((1, HIDDEN, B), lambda i, l0=l_base: (l0 + i, 0, 0))
    in_spec = pl.BlockSpec((B // 2, 2 * HIDDEN), lambda i: (i, 0))
    if out_prev is None:
        return pl.pallas_call(
            _unpack_body,
            grid=(l_cnt,),
            in_specs=[in_spec],
            out_specs=out_spec,
            out_shape=out_shape,
        )(lines_g)
    return pl.pallas_call(
        _unpack_body_alias,
        grid=(l_cnt,),
        in_specs=[in_spec, pl.BlockSpec(memory_space=pl.ANY)],
        out_specs=out_spec,
        out_shape=out_shape,
        input_output_aliases={1: 0},
    )(lines_g, out_prev)


def kernel(input_ids, emb_table, W, b):
    B, L = input_ids.shape
    n = B * L

    table128 = _transform_table(emb_table.T, W, b.reshape(1, HIDDEN))
    idx = input_ids.T.reshape(n // _CHUNK, _CHUNK).astype(jnp.int32)

    rows_per_group = n // _NGROUPS
    l_per_group = L // _NGROUPS
    out_t = None
    for g in range(_NGROUPS):
        lines_g = _make_gather(rows_per_group, g)(table128, idx)
        out_t = _unpack_group(lines_g, out_t, g * l_per_group, l_per_group, L, B)
    return jnp.transpose(out_t, (2, 0, 1))


# speedup vs baseline: 2.6700x; 1.1070x over previous
import functools

import jax
import jax.numpy as jnp
from jax import lax
from jax.experimental import pallas as pl
from jax.experimental.pallas import tpu as pltpu
from jax.experimental.pallas import tpu_sc as plsc

VOCAB = 1000000
HIDDEN = 64


_T_BLK = 16384


def _transform_body(xt_ref, w_ref, b_ref, o_ref):
    y = lax.dot_general(
        xt_ref[...],
        w_ref[...],
        dimension_numbers=(((0,), (1,)), ((), ())),
        preferred_element_type=jnp.float32,
    ) + b_ref[...]
    o_ref[:, :HIDDEN] = y


def _transform_table(emb_table_t, W, b_row):
    return pl.pallas_call(
        _transform_body,
        grid=(pl.cdiv(VOCAB, _T_BLK),),
        in_specs=[
            pl.BlockSpec((HIDDEN, _T_BLK), lambda i: (0, i)),
            pl.BlockSpec((HIDDEN, HIDDEN), lambda i: (0, 0)),
            pl.BlockSpec((1, HIDDEN), lambda i: (0, 0)),
        ],
        out_specs=pl.BlockSpec((_T_BLK, 2 * HIDDEN), lambda i: (i, 0)),
        out_shape=jax.ShapeDtypeStruct((VOCAB, 2 * HIDDEN), jnp.float32),
    )(emb_table_t, W, b_row)


_NW = 32
_NC = 2
_CHUNK = 128
_NBUF = 2
_NGROUPS = 5


def _make_gather(n_rows, group):
    n_chunks_total = n_rows // _CHUNK
    chunks_per_w = n_chunks_total // _NW
    group_base = group * n_chunks_total
    mesh = plsc.VectorSubcoreMesh(core_axis_name="c", subcore_axis_name="s")

    @functools.partial(
        pl.kernel,
        mesh=mesh,
        out_type=jax.ShapeDtypeStruct((n_rows // 2, 2 * HIDDEN), jnp.float32),
        scratch_types=[
            pltpu.VMEM((chunks_per_w, _CHUNK), jnp.int32),
            pltpu.VMEM((_NBUF, _CHUNK, 2 * HIDDEN), jnp.float32),
            pltpu.VMEM((_NBUF, _CHUNK // 2, 2 * HIDDEN), jnp.float32),
            pltpu.SemaphoreType.DMA,
            pltpu.SemaphoreType.DMA,
        ],
    )
    def gather_k(table_hbm, idx_hbm, out_hbm, idx_v, rows_v, comp_v, g_sem, s_sem):
        wid = lax.axis_index("s") * _NC + lax.axis_index("c")
        base_chunk = wid * chunks_per_w
        pltpu.sync_copy(
            idx_hbm.at[pl.ds(group_base + base_chunk, chunks_per_w)], idx_v
        )

        def compact(t):
            for u in range(_CHUNK // 2):
                for j in range(HIDDEN // 16):
                    comp_v[t, u, pl.ds(16 * j, 16)] = rows_v[
                        t, u, pl.ds(16 * j, 16)
                    ]
                    comp_v[t, u, pl.ds(HIDDEN + 16 * j, 16)] = rows_v[
                        t, 64 + u, pl.ds(16 * j, 16)
                    ]

        def block(g, _):
            gcps = []
            for t in range(_NBUF):
                gcps.append(
                    pltpu.async_copy(
                        table_hbm.at[idx_v.at[g + t]], rows_v.at[t], g_sem
                    )
                )
            scps = []
            for t in range(_NBUF):
                gcps[t].wait()
                compact(t)
                scps.append(
                    pltpu.async_copy(
                        comp_v.at[t],
                        out_hbm.at[
                            pl.ds((base_chunk + g + t) * (_CHUNK // 2), _CHUNK // 2)
                        ],
                        s_sem,
                    )
                )
            for t in range(_NBUF):
                scps[t].wait()
            return 0

        lax.fori_loop(0, chunks_per_w // _NBUF, lambda i, c: block(i * _NBUF, c), 0)

    return gather_k


def _unpack_write(x_ref, o_ref):
    t = jnp.transpose(x_ref[...])
    for c in range(x_ref.shape[0] // (_CHUNK // 2)):
        o_ref[0, :, pl.ds(_CHUNK * c, 64)] = t[:HIDDEN, 64 * c : 64 * c + 64]
        o_ref[0, :, pl.ds(_CHUNK * c + 64, 64)] = t[HIDDEN:, 64 * c : 64 * c + 64]


def _unpack_body(x_ref, o_ref):
    _unpack_write(x_ref, o_ref)


def _unpack_body_alias(x_ref, _, o_ref):
    _unpack_write(x_ref, o_ref)


def _unpack_group(lines_g, out_prev, l_base, l_cnt, L, B):
    out_shape = jax.ShapeDtypeStruct((L, HIDDEN, B), jnp.float32)
    out_spec = pl.BlockSpec((1, HIDDEN, B), lambda i, l0=l_base: (l0 + i, 0, 0))
    in_spec = pl.BlockSpec((B // 2, 2 * HIDDEN), lambda i: (i, 0))
    if out_prev is None:
        return pl.pallas_call(
            _unpack_body,
            grid=(l_cnt,),
            in_specs=[in_spec],
            out_specs=out_spec,
            out_shape=out_shape,
        )(lines_g)
    return pl.pallas_call(
        _unpack_body_alias,
        grid=(l_cnt,),
        in_specs=[in_spec, pl.BlockSpec(memory_space=pl.ANY)],
        out_specs=out_spec,
        out_shape=out_shape,
        input_output_aliases={1: 0},
    )(lines_g, out_prev)


def kernel(input_ids, emb_table, W, b):
    B, L = input_ids.shape
    n = B * L

    table128 = _transform_table(emb_table.T, W, b.reshape(1, HIDDEN))
    idx = input_ids.T.reshape(n // _CHUNK, _CHUNK).astype(jnp.int32)

    rows_per_group = n // _NGROUPS
    l_per_group = L // _NGROUPS
    out_t = None
    for g in range(_NGROUPS):
        lines_g = _make_gather(rows_per_group, g)(table128, idx)
        out_t = _unpack_group(lines_g, out_t, g * l_per_group, l_per_group, L, B)
    return jnp.transpose(out_t, (2, 0, 1))


# speedup vs baseline: 2.6822x; 1.0046x over previous
import functools

import jax
import jax.numpy as jnp
from jax import lax
from jax.experimental import pallas as pl
from jax.experimental.pallas import tpu as pltpu
from jax.experimental.pallas import tpu_sc as plsc

VOCAB = 1000000
HIDDEN = 64


_T_BLK = 16384


def _transform_body(xt_ref, w_ref, b_ref, o_ref):
    y = lax.dot_general(
        xt_ref[...],
        w_ref[...],
        dimension_numbers=(((0,), (1,)), ((), ())),
        preferred_element_type=jnp.float32,
    ) + b_ref[...]
    o_ref[:, :HIDDEN] = y


def _transform_table(emb_table_t, W, b_row):
    return pl.pallas_call(
        _transform_body,
        grid=(pl.cdiv(VOCAB, _T_BLK),),
        in_specs=[
            pl.BlockSpec((HIDDEN, _T_BLK), lambda i: (0, i)),
            pl.BlockSpec((HIDDEN, HIDDEN), lambda i: (0, 0)),
            pl.BlockSpec((1, HIDDEN), lambda i: (0, 0)),
        ],
        out_specs=pl.BlockSpec((_T_BLK, 2 * HIDDEN), lambda i: (i, 0)),
        out_shape=jax.ShapeDtypeStruct((VOCAB, 2 * HIDDEN), jnp.float32),
    )(emb_table_t, W, b_row)


_NW = 32
_NC = 2
_CHUNK = 128
_NBUF = 2
_NGROUPS = 5


def _make_gather(n_rows, group):
    n_chunks_total = n_rows // _CHUNK
    chunks_per_w = n_chunks_total // _NW
    group_base = group * n_chunks_total
    mesh = plsc.VectorSubcoreMesh(core_axis_name="c", subcore_axis_name="s")

    @functools.partial(
        pl.kernel,
        mesh=mesh,
        out_type=jax.ShapeDtypeStruct((n_rows // 2, 2 * HIDDEN), jnp.float32),
        scratch_types=[
            pltpu.VMEM((chunks_per_w, _CHUNK), jnp.int32),
            pltpu.VMEM((_NBUF, _CHUNK, 2 * HIDDEN), jnp.float32),
            pltpu.VMEM((_NBUF, _CHUNK // 2, 2 * HIDDEN), jnp.float32),
            pltpu.SemaphoreType.DMA,
            pltpu.SemaphoreType.DMA,
        ],
    )
    def gather_k(table_hbm, idx_hbm, out_hbm, idx_v, rows_v, comp_v, g_sem, s_sem):
        wid = lax.axis_index("s") * _NC + lax.axis_index("c")
        base_chunk = wid * chunks_per_w
        pltpu.sync_copy(
            idx_hbm.at[pl.ds(group_base + base_chunk, chunks_per_w)], idx_v
        )

        def compact(t):
            for u in range(_CHUNK // 2):
                for j in range(HIDDEN // 16):
                    comp_v[t, u, pl.ds(16 * j, 16)] = rows_v[
                        t, u, pl.ds(16 * j, 16)
                    ]
                    comp_v[t, u, pl.ds(HIDDEN + 16 * j, 16)] = rows_v[
                        t, 64 + u, pl.ds(16 * j, 16)
                    ]

        def block(i, _):
            g = i * _NBUF

            @pl.when(i > 0)
            def _():
                for t in range(_NBUF):
                    pltpu.make_async_copy(
                        comp_v.at[t],
                        out_hbm.at[pl.ds(0, _CHUNK // 2)],
                        s_sem,
                    ).wait()

            gcps = []
            for t in range(_NBUF):
                gcps.append(
                    pltpu.async_copy(
                        table_hbm.at[idx_v.at[g + t]], rows_v.at[t], g_sem
                    )
                )
            for t in range(_NBUF):
                gcps[t].wait()
                compact(t)
                pltpu.async_copy(
                    comp_v.at[t],
                    out_hbm.at[
                        pl.ds((base_chunk + g + t) * (_CHUNK // 2), _CHUNK // 2)
                    ],
                    s_sem,
                )
            return 0

        lax.fori_loop(0, chunks_per_w // _NBUF, block, 0)
        for t in range(_NBUF):
            pltpu.make_async_copy(
                comp_v.at[t], out_hbm.at[pl.ds(0, _CHUNK // 2)], s_sem
            ).wait()

    return gather_k


def _unpack_write(x_ref, o_ref):
    t = jnp.transpose(x_ref[...])
    for c in range(x_ref.shape[0] // (_CHUNK // 2)):
        o_ref[0, :, pl.ds(_CHUNK * c, 64)] = t[:HIDDEN, 64 * c : 64 * c + 64]
        o_ref[0, :, pl.ds(_CHUNK * c + 64, 64)] = t[HIDDEN:, 64 * c : 64 * c + 64]


def _unpack_body(x_ref, o_ref):
    _unpack_write(x_ref, o_ref)


def _unpack_body_alias(x_ref, _, o_ref):
    _unpack_write(x_ref, o_ref)


def _unpack_group(lines_g, out_prev, l_base, l_cnt, L, B):
    out_shape = jax.ShapeDtypeStruct((L, HIDDEN, B), jnp.float32)
    out_spec = pl.BlockSpec((1, HIDDEN, B), lambda i, l0=l_base: (l0 + i, 0, 0))
    in_spec = pl.BlockSpec((B // 2, 2 * HIDDEN), lambda i: (i, 0))
    if out_prev is None:
        return pl.pallas_call(
            _unpack_body,
            grid=(l_cnt,),
            in_specs=[in_spec],
            out_specs=out_spec,
            out_shape=out_shape,
        )(lines_g)
    return pl.pallas_call(
        _unpack_body_alias,
        grid=(l_cnt,),
        in_specs=[in_spec, pl.BlockSpec(memory_space=pl.ANY)],
        out_specs=out_spec,
        out_shape=out_shape,
        input_output_aliases={1: 0},
    )(lines_g, out_prev)


def kernel(input_ids, emb_table, W, b):
    B, L = input_ids.shape
    n = B * L

    table128 = _transform_table(emb_table.T, W, b.reshape(1, HIDDEN))
    idx = input_ids.T.reshape(n // _CHUNK, _CHUNK).astype(jnp.int32)

    rows_per_group = n // _NGROUPS
    l_per_group = L // _NGROUPS
    out_t = None
    for g in range(_NGROUPS):
        lines_g = _make_gather(rows_per_group, g)(table128, idx)
        out_t = _unpack_group(lines_g, out_t, g * l_per_group, l_per_group, L, B)
    return jnp.transpose(out_t, (2, 0, 1))
